# baseline (speedup 1.0000x reference)
import functools

import jax
import jax.numpy as jnp
from jax import lax
from jax.experimental import pallas as pl
from jax.experimental.pallas import tpu as pltpu
from jax.experimental.pallas import tpu_sc as plsc

N = 10000
E = 320000
DIN = 128
DH = 256
NCLS = 2
EPS = 1e-5

EC = 128
ER = E // EC
NPAD = 10240
NTILES = 16
NCORES = 2


def _fill_vec(ref, n16, value):
    def body(i, _):
        ref[pl.ds(i * 16, 16)] = jnp.full((16,), value, jnp.float32)
        return 0
    lax.fori_loop(0, n16, body, 0)


def _deg_body(ei_hbm, hist_out, ones_v, zeros_v, idx2_v, hsrc, hdst, sem):
    c = lax.axis_index("c")
    s = lax.axis_index("s")
    _fill_vec(ones_v, EC // 16, 1.0)
    _fill_vec(zeros_v, 640 // 16, 0.0)
    pltpu.sync_copy(zeros_v, hsrc.at[pl.ds(s * 640, 640)])
    pltpu.sync_copy(zeros_v, hdst.at[pl.ds(s * 640, 640)])
    plsc.subcore_barrier()
    def body(i, _):
        r = s + i * NTILES

        @pl.when(r < ER // NCORES)
        def _():
            row = c * (ER // NCORES) + r
            pltpu.sync_copy(ei_hbm.at[:, row], idx2_v)
            pltpu.sync_copy(ones_v, hsrc.at[idx2_v.at[0]], add=True)
            pltpu.sync_copy(ones_v, hdst.at[idx2_v.at[1]], add=True)
        return 0
    lax.fori_loop(0, (ER // NCORES + NTILES - 1) // NTILES, body, 0)
    plsc.subcore_barrier()
    pltpu.sync_copy(hsrc.at[pl.ds(s * 640, 640)], hist_out.at[c, 0, pl.ds(s * 640, 640)])
    pltpu.sync_copy(hdst.at[pl.ds(s * 640, 640)], hist_out.at[c, 1, pl.ds(s * 640, 640)])


def _make_deg_kernel():
    mesh = plsc.VectorSubcoreMesh(core_axis_name="c", subcore_axis_name="s")
    return functools.partial(
        pl.kernel,
        mesh=mesh,
        out_type=jax.ShapeDtypeStruct((NCORES, 2, NPAD), jnp.float32),
        scratch_types=[
            pltpu.VMEM((EC,), jnp.float32),
            pltpu.VMEM((640,), jnp.float32),
            pltpu.VMEM((2, EC), jnp.int32),
            pltpu.VMEM_SHARED((NPAD,), jnp.float32),
            pltpu.VMEM_SHARED((NPAD,), jnp.float32),
            pltpu.SemaphoreType.DMA,
        ],
    )(_deg_body)


HALF = 128


def _agg_body(edge_split, h0_hbm, h1_hbm, ei_hbm, zero_hbm, agg0_out, agg1_out,
              idx2_v, rows_v, acc,
              isem0, isem1, isem2, gsem0, gsem1, gsem2, ssem0, ssem1, ssem2):
    isem = (isem0, isem1, isem2)
    gsem = (gsem0, gsem1, gsem2)
    ssem = (ssem0, ssem1, ssem2)
    c = lax.axis_index("c")
    s = lax.axis_index("s")
    def zacc(j, _):
        pltpu.sync_copy(zero_hbm, acc.at[pl.ds(s * 624 + j * 64, 64)])
        return 0
    lax.fori_loop(0, 9, zacc, 0)
    pltpu.sync_copy(zero_hbm.at[pl.ds(0, 56)],
                    acc.at[pl.ds(s * 624 + 576, 56)])

    @pl.when(s == NTILES - 1)
    def _():
        pltpu.sync_copy(zero_hbm.at[pl.ds(0, 8)], acc.at[pl.ds(N - 8, 8)])
    plsc.subcore_barrier()

    rpc = ER // NCORES if edge_split else ER
    niter = (rpc + NTILES - 1) // NTILES

    def vld(i):
        return jnp.logical_and(i >= 0, s + i * NTILES < rpc)

    def issue_idx(i, b):
        r = s + i * NTILES
        row = c * rpc + r if edge_split else r
        pltpu.async_copy(ei_hbm.at[:, row], idx2_v.at[b], isem[b])

    def wait_idx(b):
        pltpu.make_async_copy(ei_hbm.at[:, 0], idx2_v.at[b], isem[b]).wait()

    def issue_gather(b):
        if edge_split:
            pltpu.async_copy(h0_hbm.at[idx2_v.at[b, 0]], rows_v.at[b], gsem[b])
        else:
            @pl.when(c == 0)
            def _():
                pltpu.async_copy(h0_hbm.at[idx2_v.at[b, 0]], rows_v.at[b],
                                 gsem[b])

            @pl.when(c == 1)
            def _():
                pltpu.async_copy(h1_hbm.at[idx2_v.at[b, 0]], rows_v.at[b],
                                 gsem[b])

    def wait_gather(b):
        pltpu.make_async_copy(h0_hbm.at[idx2_v.at[b, 0]], rows_v.at[b],
                              gsem[b]).wait()

    def start_scatter(b):
        pltpu.make_async_copy(rows_v.at[b], acc.at[idx2_v.at[b, 1]],
                              ssem[b]).start(add=True)

    def wait_scatter(b):
        pltpu.make_async_copy(rows_v.at[b], acc.at[idx2_v.at[b, 1]],
                              ssem[b]).wait()

    for i0 in (0, 1):
        @pl.when(vld(i0))
        def _():
            issue_idx(i0, i0)

    @pl.when(vld(0))
    def _():
        wait_idx(0)
        issue_gather(0)

    def tri_body(ip, _):
        for b in (0, 1, 2):
            i = ip * 3 + b
            b1 = (b + 1) % 3
            b2 = (b + 2) % 3

            @pl.when(vld(i))
            def _():
                wait_gather(b)
                start_scatter(b)

            @pl.when(vld(i + 1))
            def _():
                wait_idx(b1)
                issue_gather(b1)

            @pl.when(vld(i + 2))
            def _():
                @pl.when(vld(i - 1))
                def _():
                    wait_scatter(b2)
                issue_idx(i + 2, b2)
        return 0
    lax.fori_loop(0, (niter + 2) // 3, tri_body, 0)

    for i0 in (1, 2, 3, 4):
        k = niter - i0
        if k >= 0:
            @pl.when(jnp.logical_and(vld(k), jnp.logical_not(vld(k + 3))))
            def _():
                wait_scatter(k % 3)
    plsc.subcore_barrier()

    @pl.when(c == 0)
    def _():
        pltpu.sync_copy(acc.at[pl.ds(s * 624, 632)], agg0_out.at[pl.ds(s * 624, 632)])

        @pl.when(s == NTILES - 1)
        def _():
            pltpu.sync_copy(acc.at[pl.ds(N - 8, 8)], agg0_out.at[pl.ds(N - 8, 8)])

    @pl.when(c == 1)
    def _():
        pltpu.sync_copy(acc.at[pl.ds(s * 624, 632)], agg1_out.at[pl.ds(s * 624, 632)])

        @pl.when(s == NTILES - 1)
        def _():
            pltpu.sync_copy(acc.at[pl.ds(N - 8, 8)], agg1_out.at[pl.ds(N - 8, 8)])


def _make_agg_kernel(edge_split):
    mesh = plsc.VectorSubcoreMesh(core_axis_name="c", subcore_axis_name="s")
    return functools.partial(
        pl.kernel,
        mesh=mesh,
        out_type=(
            jax.ShapeDtypeStruct((NPAD, HALF), jnp.float32),
            jax.ShapeDtypeStruct((NPAD, HALF), jnp.float32),
        ),
        scratch_types=[
            pltpu.VMEM((3, 2, EC), jnp.int32),
            pltpu.VMEM((3, EC, HALF), jnp.float32),
            pltpu.VMEM_SHARED((N, HALF), jnp.float32),
        ] + [pltpu.SemaphoreType.DMA] * 9,
    )(functools.partial(_agg_body, edge_split))


def _prep_body(x_ref, hist_ref, hs_ref, degv_ref):
    deg_out = hist_ref[0, 0, :] + hist_ref[1, 0, :]
    deg_in = hist_ref[0, 1, :] + hist_ref[1, 1, :]
    dinv_out = jax.lax.rsqrt(jnp.maximum(deg_out, 1.0))
    dinv_in = jax.lax.rsqrt(jnp.maximum(deg_in, 1.0))
    degv_ref[0, :] = dinv_out
    degv_ref[1, :] = dinv_in
    hs_ref[...] = x_ref[...] * dinv_out[:N, None]


def _tc_prep(features, hist):
    return pl.pallas_call(
        _prep_body,
        out_shape=(
            jax.ShapeDtypeStruct((N, DIN), jnp.float32),
            jax.ShapeDtypeStruct((2, NPAD), jnp.float32),
        ),
    )(features, hist)


BLK = 1024
NBLK = NPAD // BLK


def _mm_body(sum_mode, a0_ref, a1_ref, degv_ref, w_ref, b_ref, t_ref, stats_ref):
    i = pl.program_id(0)
    if sum_mode:
        a = a0_ref[...] + a1_ref[...]
    else:
        a = jnp.concatenate([a0_ref[...], a1_ref[...]], axis=1)
    din = degv_ref[1, pl.ds(i * BLK, BLK)]
    a = a * din[:, None]
    t = jnp.dot(a, w_ref[...], preferred_element_type=jnp.float32, precision=jax.lax.Precision.HIGHEST) + b_ref[...]
    t_ref[...] = t

    @pl.when(i == 0)
    def _():
        stats_ref[...] = jnp.zeros_like(stats_ref)
    rows = jax.lax.broadcasted_iota(jnp.int32, (BLK, 1), 0) + i * BLK
    tm = jnp.where(rows < N, t, 0.0)
    stats_ref[0, :] += jnp.sum(tm, axis=0)
    stats_ref[1, :] += jnp.sum(tm * tm, axis=0)


def _tc_matmul(agg0, agg1, degv, W, b, sum_mode):
    din, dout = W.shape
    return pl.pallas_call(
        functools.partial(_mm_body, sum_mode),
        grid=(NBLK,),
        in_specs=[
            pl.BlockSpec((BLK, HALF), lambda i: (i, 0)),
            pl.BlockSpec((BLK, HALF), lambda i: (i, 0)),
            pl.BlockSpec((2, NPAD), lambda i: (0, 0)),
            pl.BlockSpec((din, dout), lambda i: (0, 0)),
            pl.BlockSpec((dout,), lambda i: (0,)),
        ],
        out_specs=(
            pl.BlockSpec((BLK, dout), lambda i: (i, 0)),
            pl.BlockSpec((2, dout), lambda i: (0, 0)),
        ),
        out_shape=(
            jax.ShapeDtypeStruct((NPAD, dout), jnp.float32),
            jax.ShapeDtypeStruct((2, dout), jnp.float32),
        ),
    )(agg0, agg1, degv, W, b)


def _bn_body(t_ref, stats_ref, g_ref, be_ref, degv_ref, h0_ref, h1_ref):
    i = pl.program_id(0)
    mean = stats_ref[0, :] * (1.0 / N)
    var = stats_ref[1, :] * (1.0 / N) - mean * mean
    inv = jax.lax.rsqrt(var + EPS)
    y = (t_ref[...] - mean[None, :]) * (inv * g_ref[...])[None, :] + be_ref[...][None, :]
    y = jnp.maximum(y, 0.0)
    dout = degv_ref[0, pl.ds(i * BLK, BLK)]
    y = y * dout[:, None]
    h0_ref[...] = y[:, : DH // 2]
    h1_ref[...] = y[:, DH // 2:]


def _tc_bn_split(t, stats, g, be, degv):
    return pl.pallas_call(
        _bn_body,
        grid=(NBLK,),
        in_specs=[
            pl.BlockSpec((BLK, DH), lambda i: (i, 0)),
            pl.BlockSpec((2, DH), lambda i: (0, 0)),
            pl.BlockSpec((DH,), lambda i: (0,)),
            pl.BlockSpec((DH,), lambda i: (0,)),
            pl.BlockSpec((2, NPAD), lambda i: (0, 0)),
        ],
        out_specs=(
            pl.BlockSpec((BLK, DH // 2), lambda i: (i, 0)),
            pl.BlockSpec((BLK, DH // 2), lambda i: (i, 0)),
        ),
        out_shape=(
            jax.ShapeDtypeStruct((NPAD, DH // 2), jnp.float32),
            jax.ShapeDtypeStruct((NPAD, DH // 2), jnp.float32),
        ),
    )(t, stats, g, be, degv)


def _cls_body(t_ref, stats_ref, g_ref, be_ref, wc_ref, bc_ref, o_ref):
    mean = stats_ref[0, :] * (1.0 / N)
    var = stats_ref[1, :] * (1.0 / N) - mean * mean
    inv = jax.lax.rsqrt(var + EPS)
    y = (t_ref[...] - mean[None, :]) * (inv * g_ref[...])[None, :] + be_ref[...][None, :]
    y = jnp.maximum(y, 0.0)
    o_ref[...] = jnp.dot(y, wc_ref[...], preferred_element_type=jnp.float32, precision=jax.lax.Precision.HIGHEST) + bc_ref[...]


def _tc_classifier(t, stats, g, be, wc_pad, bc_pad):
    return pl.pallas_call(
        _cls_body,
        grid=(NBLK,),
        in_specs=[
            pl.BlockSpec((BLK, DH), lambda i: (i, 0)),
            pl.BlockSpec((2, DH), lambda i: (0, 0)),
            pl.BlockSpec((DH,), lambda i: (0,)),
            pl.BlockSpec((DH,), lambda i: (0,)),
            pl.BlockSpec((DH, 128), lambda i: (0, 0)),
            pl.BlockSpec((128,), lambda i: (0,)),
        ],
        out_specs=pl.BlockSpec((BLK, 128), lambda i: (i, 0)),
        out_shape=jax.ShapeDtypeStruct((NPAD, 128), jnp.float32),
    )(t, stats, g, be, wc_pad, bc_pad)


def kernel(features, edge_index, W1, b1, g1, be1, W2, b2, g2, be2,
           W3, b3, g3, be3, Wc, bc):
    ei = edge_index.reshape(2, ER, EC)

    hist = _make_deg_kernel()(ei)
    hs, degv = _tc_prep(features, hist)
    zer = jnp.zeros((64, HALF), jnp.float32)

    agg0, agg1 = _make_agg_kernel(True)(hs, hs, ei, zer)
    t1, st1 = _tc_matmul(agg0, agg1, degv, W1, b1, True)
    h0, h1 = _tc_bn_split(t1, st1, g1, be1, degv)

    agg0, agg1 = _make_agg_kernel(False)(h0, h1, ei, zer)
    t2, st2 = _tc_matmul(agg0, agg1, degv, W2, b2, False)
    h0, h1 = _tc_bn_split(t2, st2, g2, be2, degv)

    agg0, agg1 = _make_agg_kernel(False)(h0, h1, ei, zer)
    t3, st3 = _tc_matmul(agg0, agg1, degv, W3, b3, False)

    wc_pad = jnp.zeros((DH, 128), jnp.float32).at[:, :NCLS].set(Wc)
    bc_pad = jnp.zeros((128,), jnp.float32).at[:NCLS].set(bc)
    out = _tc_classifier(t3, st3, g3, be3, wc_pad, bc_pad)
    return out[:N, :NCLS]


# speedup vs baseline: 1.0502x; 1.0502x over previous
import functools

import jax
import jax.numpy as jnp
from jax import lax
from jax.experimental import pallas as pl
from jax.experimental.pallas import tpu as pltpu
from jax.experimental.pallas import tpu_sc as plsc

N = 10000
E = 320000
DIN = 128
DH = 256
NCLS = 2
EPS = 1e-5

EC = 128
ER = E // EC
NPAD = 10240
NTILES = 16
NCORES = 2


def _fill_vec(ref, n16, value):
    def body(i, _):
        ref[pl.ds(i * 16, 16)] = jnp.full((16,), value, jnp.float32)
        return 0
    lax.fori_loop(0, n16, body, 0)


def _deg_body(ei_hbm, hist_out, ones_v, zeros_v, idx2_v, hsrc, hdst, sem):
    c = lax.axis_index("c")
    s = lax.axis_index("s")
    _fill_vec(ones_v, EC // 16, 1.0)
    _fill_vec(zeros_v, 640 // 16, 0.0)
    pltpu.sync_copy(zeros_v, hsrc.at[pl.ds(s * 640, 640)])
    pltpu.sync_copy(zeros_v, hdst.at[pl.ds(s * 640, 640)])
    plsc.subcore_barrier()
    def body(i, _):
        r = s + i * NTILES

        @pl.when(r < ER // NCORES)
        def _():
            row = c * (ER // NCORES) + r
            pltpu.sync_copy(ei_hbm.at[:, row], idx2_v)
            pltpu.sync_copy(ones_v, hsrc.at[idx2_v.at[0]], add=True)
            pltpu.sync_copy(ones_v, hdst.at[idx2_v.at[1]], add=True)
        return 0
    lax.fori_loop(0, (ER // NCORES + NTILES - 1) // NTILES, body, 0)
    plsc.subcore_barrier()
    pltpu.sync_copy(hsrc.at[pl.ds(s * 640, 640)], hist_out.at[c, 0, pl.ds(s * 640, 640)])
    pltpu.sync_copy(hdst.at[pl.ds(s * 640, 640)], hist_out.at[c, 1, pl.ds(s * 640, 640)])


def _make_deg_kernel():
    mesh = plsc.VectorSubcoreMesh(core_axis_name="c", subcore_axis_name="s")
    return functools.partial(
        pl.kernel,
        mesh=mesh,
        out_type=jax.ShapeDtypeStruct((NCORES, 2, NPAD), jnp.float32),
        scratch_types=[
            pltpu.VMEM((EC,), jnp.float32),
            pltpu.VMEM((640,), jnp.float32),
            pltpu.VMEM((2, EC), jnp.int32),
            pltpu.VMEM_SHARED((NPAD,), jnp.float32),
            pltpu.VMEM_SHARED((NPAD,), jnp.float32),
            pltpu.SemaphoreType.DMA,
        ],
    )(_deg_body)


HALF = 128


def _agg_body(edge_split, h0_hbm, h1_hbm, ei_hbm, zero_hbm, agg0_out, agg1_out,
              idx2_v, rows_v, acc,
              isem0, isem1, isem2, gsem0, gsem1, gsem2, ssem0, ssem1, ssem2):
    isem = (isem0, isem1, isem2)
    gsem = (gsem0, gsem1, gsem2)
    ssem = (ssem0, ssem1, ssem2)
    c = lax.axis_index("c")
    s = lax.axis_index("s")
    pltpu.sync_copy(zero_hbm.at[pl.ds(0, 632)],
                    acc.at[pl.ds(s * 624, 632)])

    @pl.when(s == NTILES - 1)
    def _():
        pltpu.sync_copy(zero_hbm.at[pl.ds(0, 8)], acc.at[pl.ds(N - 8, 8)])
    plsc.subcore_barrier()

    rpc = ER // NCORES if edge_split else ER
    niter = (rpc + NTILES - 1) // NTILES

    def vld(i):
        return jnp.logical_and(i >= 0, s + i * NTILES < rpc)

    def issue_idx(i, b):
        r = s + i * NTILES
        row = c * rpc + r if edge_split else r
        pltpu.async_copy(ei_hbm.at[:, row], idx2_v.at[b], isem[b])

    def wait_idx(b):
        pltpu.make_async_copy(ei_hbm.at[:, 0], idx2_v.at[b], isem[b]).wait()

    def issue_gather(b):
        if edge_split:
            pltpu.async_copy(h0_hbm.at[idx2_v.at[b, 0]], rows_v.at[b], gsem[b])
        else:
            @pl.when(c == 0)
            def _():
                pltpu.async_copy(h0_hbm.at[idx2_v.at[b, 0]], rows_v.at[b],
                                 gsem[b])

            @pl.when(c == 1)
            def _():
                pltpu.async_copy(h1_hbm.at[idx2_v.at[b, 0]], rows_v.at[b],
                                 gsem[b])

    def wait_gather(b):
        pltpu.make_async_copy(h0_hbm.at[idx2_v.at[b, 0]], rows_v.at[b],
                              gsem[b]).wait()

    def start_scatter(b):
        pltpu.make_async_copy(rows_v.at[b], acc.at[idx2_v.at[b, 1]],
                              ssem[b]).start(add=True)

    def wait_scatter(b):
        pltpu.make_async_copy(rows_v.at[b], acc.at[idx2_v.at[b, 1]],
                              ssem[b]).wait()

    for i0 in (0, 1):
        @pl.when(vld(i0))
        def _():
            issue_idx(i0, i0)

    @pl.when(vld(0))
    def _():
        wait_idx(0)
        issue_gather(0)

    def tri_body(ip, _):
        for b in (0, 1, 2):
            i = ip * 3 + b
            b1 = (b + 1) % 3
            b2 = (b + 2) % 3

            @pl.when(vld(i))
            def _():
                wait_gather(b)
                start_scatter(b)

            @pl.when(vld(i + 1))
            def _():
                wait_idx(b1)
                issue_gather(b1)

            @pl.when(vld(i + 2))
            def _():
                @pl.when(vld(i - 1))
                def _():
                    wait_scatter(b2)
                issue_idx(i + 2, b2)
        return 0
    lax.fori_loop(0, (niter + 2) // 3, tri_body, 0)

    for i0 in (1, 2, 3, 4):
        k = niter - i0
        if k >= 0:
            @pl.when(jnp.logical_and(vld(k), jnp.logical_not(vld(k + 3))))
            def _():
                wait_scatter(k % 3)
    plsc.subcore_barrier()

    @pl.when(c == 0)
    def _():
        pltpu.sync_copy(acc.at[pl.ds(s * 624, 632)], agg0_out.at[pl.ds(s * 624, 632)])

        @pl.when(s == NTILES - 1)
        def _():
            pltpu.sync_copy(acc.at[pl.ds(N - 8, 8)], agg0_out.at[pl.ds(N - 8, 8)])

    @pl.when(c == 1)
    def _():
        pltpu.sync_copy(acc.at[pl.ds(s * 624, 632)], agg1_out.at[pl.ds(s * 624, 632)])

        @pl.when(s == NTILES - 1)
        def _():
            pltpu.sync_copy(acc.at[pl.ds(N - 8, 8)], agg1_out.at[pl.ds(N - 8, 8)])


def _make_agg_kernel(edge_split):
    mesh = plsc.VectorSubcoreMesh(core_axis_name="c", subcore_axis_name="s")
    return functools.partial(
        pl.kernel,
        mesh=mesh,
        out_type=(
            jax.ShapeDtypeStruct((NPAD, HALF), jnp.float32),
            jax.ShapeDtypeStruct((NPAD, HALF), jnp.float32),
        ),
        scratch_types=[
            pltpu.VMEM((3, 2, EC), jnp.int32),
            pltpu.VMEM((3, EC, HALF), jnp.float32),
            pltpu.VMEM_SHARED((N, HALF), jnp.float32),
        ] + [pltpu.SemaphoreType.DMA] * 9,
    )(functools.partial(_agg_body, edge_split))


def _prep_body(x_ref, hist_ref, hs_ref, degv_ref):
    deg_out = hist_ref[0, 0, :] + hist_ref[1, 0, :]
    deg_in = hist_ref[0, 1, :] + hist_ref[1, 1, :]
    dinv_out = jax.lax.rsqrt(jnp.maximum(deg_out, 1.0))
    dinv_in = jax.lax.rsqrt(jnp.maximum(deg_in, 1.0))
    degv_ref[0, :] = dinv_out
    degv_ref[1, :] = dinv_in
    hs_ref[...] = x_ref[...] * dinv_out[:N, None]


def _tc_prep(features, hist):
    return pl.pallas_call(
        _prep_body,
        out_shape=(
            jax.ShapeDtypeStruct((N, DIN), jnp.float32),
            jax.ShapeDtypeStruct((2, NPAD), jnp.float32),
        ),
    )(features, hist)


BLK = 1024
NBLK = NPAD // BLK


def _mm_body(sum_mode, a0_ref, a1_ref, degv_ref, w_ref, b_ref, t_ref, stats_ref):
    i = pl.program_id(0)
    if sum_mode:
        a = a0_ref[...] + a1_ref[...]
    else:
        a = jnp.concatenate([a0_ref[...], a1_ref[...]], axis=1)
    din = degv_ref[1, pl.ds(i * BLK, BLK)]
    a = a * din[:, None]
    t = jnp.dot(a, w_ref[...], preferred_element_type=jnp.float32, precision=jax.lax.Precision.HIGHEST) + b_ref[...]
    t_ref[...] = t

    @pl.when(i == 0)
    def _():
        stats_ref[...] = jnp.zeros_like(stats_ref)
    rows = jax.lax.broadcasted_iota(jnp.int32, (BLK, 1), 0) + i * BLK
    tm = jnp.where(rows < N, t, 0.0)
    stats_ref[0, :] += jnp.sum(tm, axis=0)
    stats_ref[1, :] += jnp.sum(tm * tm, axis=0)


def _tc_matmul(agg0, agg1, degv, W, b, sum_mode):
    din, dout = W.shape
    return pl.pallas_call(
        functools.partial(_mm_body, sum_mode),
        grid=(NBLK,),
        in_specs=[
            pl.BlockSpec((BLK, HALF), lambda i: (i, 0)),
            pl.BlockSpec((BLK, HALF), lambda i: (i, 0)),
            pl.BlockSpec((2, NPAD), lambda i: (0, 0)),
            pl.BlockSpec((din, dout), lambda i: (0, 0)),
            pl.BlockSpec((dout,), lambda i: (0,)),
        ],
        out_specs=(
            pl.BlockSpec((BLK, dout), lambda i: (i, 0)),
            pl.BlockSpec((2, dout), lambda i: (0, 0)),
        ),
        out_shape=(
            jax.ShapeDtypeStruct((NPAD, dout), jnp.float32),
            jax.ShapeDtypeStruct((2, dout), jnp.float32),
        ),
    )(agg0, agg1, degv, W, b)


def _bn_body(t_ref, stats_ref, g_ref, be_ref, degv_ref, h0_ref, h1_ref):
    i = pl.program_id(0)
    mean = stats_ref[0, :] * (1.0 / N)
    var = stats_ref[1, :] * (1.0 / N) - mean * mean
    inv = jax.lax.rsqrt(var + EPS)
    y = (t_ref[...] - mean[None, :]) * (inv * g_ref[...])[None, :] + be_ref[...][None, :]
    y = jnp.maximum(y, 0.0)
    dout = degv_ref[0, pl.ds(i * BLK, BLK)]
    y = y * dout[:, None]
    h0_ref[...] = y[:, : DH // 2]
    h1_ref[...] = y[:, DH // 2:]


def _tc_bn_split(t, stats, g, be, degv):
    return pl.pallas_call(
        _bn_body,
        grid=(NBLK,),
        in_specs=[
            pl.BlockSpec((BLK, DH), lambda i: (i, 0)),
            pl.BlockSpec((2, DH), lambda i: (0, 0)),
            pl.BlockSpec((DH,), lambda i: (0,)),
            pl.BlockSpec((DH,), lambda i: (0,)),
            pl.BlockSpec((2, NPAD), lambda i: (0, 0)),
        ],
        out_specs=(
            pl.BlockSpec((BLK, DH // 2), lambda i: (i, 0)),
            pl.BlockSpec((BLK, DH // 2), lambda i: (i, 0)),
        ),
        out_shape=(
            jax.ShapeDtypeStruct((NPAD, DH // 2), jnp.float32),
            jax.ShapeDtypeStruct((NPAD, DH // 2), jnp.float32),
        ),
    )(t, stats, g, be, degv)


def _cls_body(t_ref, stats_ref, g_ref, be_ref, wc_ref, bc_ref, o_ref):
    mean = stats_ref[0, :] * (1.0 / N)
    var = stats_ref[1, :] * (1.0 / N) - mean * mean
    inv = jax.lax.rsqrt(var + EPS)
    y = (t_ref[...] - mean[None, :]) * (inv * g_ref[...])[None, :] + be_ref[...][None, :]
    y = jnp.maximum(y, 0.0)
    o_ref[...] = jnp.dot(y, wc_ref[...], preferred_element_type=jnp.float32, precision=jax.lax.Precision.HIGHEST) + bc_ref[...]


def _tc_classifier(t, stats, g, be, wc_pad, bc_pad):
    return pl.pallas_call(
        _cls_body,
        grid=(NBLK,),
        in_specs=[
            pl.BlockSpec((BLK, DH), lambda i: (i, 0)),
            pl.BlockSpec((2, DH), lambda i: (0, 0)),
            pl.BlockSpec((DH,), lambda i: (0,)),
            pl.BlockSpec((DH,), lambda i: (0,)),
            pl.BlockSpec((DH, 128), lambda i: (0, 0)),
            pl.BlockSpec((128,), lambda i: (0,)),
        ],
        out_specs=pl.BlockSpec((BLK, 128), lambda i: (i, 0)),
        out_shape=jax.ShapeDtypeStruct((NPAD, 128), jnp.float32),
    )(t, stats, g, be, wc_pad, bc_pad)


def kernel(features, edge_index, W1, b1, g1, be1, W2, b2, g2, be2,
           W3, b3, g3, be3, Wc, bc):
    ei = edge_index.reshape(2, ER, EC)

    hist = _make_deg_kernel()(ei)
    hs, degv = _tc_prep(features, hist)
    zer = jnp.zeros((632, HALF), jnp.float32)

    agg0, agg1 = _make_agg_kernel(True)(hs, hs, ei, zer)
    t1, st1 = _tc_matmul(agg0, agg1, degv, W1, b1, True)
    h0, h1 = _tc_bn_split(t1, st1, g1, be1, degv)

    agg0, agg1 = _make_agg_kernel(False)(h0, h1, ei, zer)
    t2, st2 = _tc_matmul(agg0, agg1, degv, W2, b2, False)
    h0, h1 = _tc_bn_split(t2, st2, g2, be2, degv)

    agg0, agg1 = _make_agg_kernel(False)(h0, h1, ei, zer)
    t3, st3 = _tc_matmul(agg0, agg1, degv, W3, b3, False)

    wc_pad = jnp.zeros((DH, 128), jnp.float32).at[:, :NCLS].set(Wc)
    bc_pad = jnp.zeros((128,), jnp.float32).at[:NCLS].set(bc)
    out = _tc_classifier(t3, st3, g3, be3, wc_pad, bc_pad)
    return out[:N, :NCLS]


# speedup vs baseline: 1.2287x; 1.1700x over previous
import functools

import jax
import jax.numpy as jnp
from jax import lax
from jax.experimental import pallas as pl
from jax.experimental.pallas import tpu as pltpu
from jax.experimental.pallas import tpu_sc as plsc

N = 10000
E = 320000
DIN = 128
DH = 256
NCLS = 2
EPS = 1e-5

EC = 128
ER = E // EC
NPAD = 10240
NTILES = 16
NCORES = 2


def _fill_vec(ref, n16, value):
    def body(i, _):
        ref[pl.ds(i * 16, 16)] = jnp.full((16,), value, jnp.float32)
        return 0
    lax.fori_loop(0, n16, body, 0)


def _deg_body(ei_hbm, hist_out, ones_v, zeros_v, idx2_v, hsrc, hdst, sem):
    c = lax.axis_index("c")
    s = lax.axis_index("s")
    _fill_vec(ones_v, EC // 16, 1.0)
    _fill_vec(zeros_v, 640 // 16, 0.0)
    pltpu.sync_copy(zeros_v, hsrc.at[pl.ds(s * 640, 640)])
    pltpu.sync_copy(zeros_v, hdst.at[pl.ds(s * 640, 640)])
    plsc.subcore_barrier()
    def body(i, _):
        r = s + i * NTILES

        @pl.when(r < ER // NCORES)
        def _():
            row = c * (ER // NCORES) + r
            pltpu.sync_copy(ei_hbm.at[:, row], idx2_v)
            pltpu.sync_copy(ones_v, hsrc.at[idx2_v.at[0]], add=True)
            pltpu.sync_copy(ones_v, hdst.at[idx2_v.at[1]], add=True)
        return 0
    lax.fori_loop(0, (ER // NCORES + NTILES - 1) // NTILES, body, 0)
    plsc.subcore_barrier()
    pltpu.sync_copy(hsrc.at[pl.ds(s * 640, 640)], hist_out.at[c, 0, pl.ds(s * 640, 640)])
    pltpu.sync_copy(hdst.at[pl.ds(s * 640, 640)], hist_out.at[c, 1, pl.ds(s * 640, 640)])


def _make_deg_kernel():
    mesh = plsc.VectorSubcoreMesh(core_axis_name="c", subcore_axis_name="s")
    return functools.partial(
        pl.kernel,
        mesh=mesh,
        out_type=jax.ShapeDtypeStruct((NCORES, 2, NPAD), jnp.float32),
        scratch_types=[
            pltpu.VMEM((EC,), jnp.float32),
            pltpu.VMEM((640,), jnp.float32),
            pltpu.VMEM((2, EC), jnp.int32),
            pltpu.VMEM_SHARED((NPAD,), jnp.float32),
            pltpu.VMEM_SHARED((NPAD,), jnp.float32),
            pltpu.SemaphoreType.DMA,
        ],
    )(_deg_body)


HALF = 128


def _agg_body(edge_split, h0_hbm, h1_hbm, ei_hbm, zero_hbm, agg0_out, agg1_out,
              idx2_v, rows_v, acc,
              isem0, isem1, isem2, gsem0, gsem1, gsem2):
    isem = (isem0, isem1, isem2)
    gsem = (gsem0, gsem1, gsem2)
    c = lax.axis_index("c")
    s = lax.axis_index("s")
    pltpu.sync_copy(zero_hbm.at[pl.ds(0, 632)],
                    acc.at[pl.ds(s * 624, 632)])

    @pl.when(s == NTILES - 1)
    def _():
        pltpu.sync_copy(zero_hbm.at[pl.ds(0, 8)], acc.at[pl.ds(N - 8, 8)])
    plsc.subcore_barrier()

    rpc = ER // NCORES if edge_split else ER
    niter = (rpc + NTILES - 1) // NTILES

    def vld(i):
        return jnp.logical_and(i >= 0, s + i * NTILES < rpc)

    def issue_idx(i, b):
        r = s + i * NTILES
        row = c * rpc + r if edge_split else r
        pltpu.async_copy(ei_hbm.at[:, row], idx2_v.at[b], isem[b])

    def wait_idx(b):
        pltpu.make_async_copy(ei_hbm.at[:, 0], idx2_v.at[b], isem[b]).wait()

    def issue_gather(b):
        if edge_split:
            pltpu.async_copy(h0_hbm.at[idx2_v.at[b, 0]], rows_v.at[b], gsem[b])
        else:
            @pl.when(c == 0)
            def _():
                pltpu.async_copy(h0_hbm.at[idx2_v.at[b, 0]], rows_v.at[b],
                                 gsem[b])

            @pl.when(c == 1)
            def _():
                pltpu.async_copy(h1_hbm.at[idx2_v.at[b, 0]], rows_v.at[b],
                                 gsem[b])

    def wait_gather(b):
        pltpu.make_async_copy(h0_hbm.at[idx2_v.at[b, 0]], rows_v.at[b],
                              gsem[b]).wait()

    for i0 in (0, 1):
        @pl.when(vld(i0))
        def _():
            issue_idx(i0, i0)

    for i0 in (0, 1):
        @pl.when(vld(i0))
        def _():
            wait_idx(i0)
            issue_gather(i0)

    @pl.when(vld(2))
    def _():
        issue_idx(2, 2)

    def tri_body(ip, _):
        for b in (0, 1, 2):
            i = ip * 3 + b
            b2 = (b + 2) % 3

            @pl.when(vld(i))
            def _():
                wait_gather(b)
                pltpu.sync_copy(rows_v.at[b], acc.at[idx2_v.at[b, 1]],
                                add=True)

                @pl.when(vld(i + 3))
                def _():
                    issue_idx(i + 3, b)

            @pl.when(vld(i + 2))
            def _():
                wait_idx(b2)
                issue_gather(b2)
        return 0
    lax.fori_loop(0, (niter + 2) // 3, tri_body, 0)
    plsc.subcore_barrier()

    @pl.when(c == 0)
    def _():
        pltpu.sync_copy(acc.at[pl.ds(s * 624, 632)], agg0_out.at[pl.ds(s * 624, 632)])

        @pl.when(s == NTILES - 1)
        def _():
            pltpu.sync_copy(acc.at[pl.ds(N - 8, 8)], agg0_out.at[pl.ds(N - 8, 8)])

    @pl.when(c == 1)
    def _():
        pltpu.sync_copy(acc.at[pl.ds(s * 624, 632)], agg1_out.at[pl.ds(s * 624, 632)])

        @pl.when(s == NTILES - 1)
        def _():
            pltpu.sync_copy(acc.at[pl.ds(N - 8, 8)], agg1_out.at[pl.ds(N - 8, 8)])


def _make_agg_kernel(edge_split):
    mesh = plsc.VectorSubcoreMesh(core_axis_name="c", subcore_axis_name="s")
    return functools.partial(
        pl.kernel,
        mesh=mesh,
        out_type=(
            jax.ShapeDtypeStruct((NPAD, HALF), jnp.float32),
            jax.ShapeDtypeStruct((NPAD, HALF), jnp.float32),
        ),
        scratch_types=[
            pltpu.VMEM((3, 2, EC), jnp.int32),
            pltpu.VMEM((3, EC, HALF), jnp.float32),
            pltpu.VMEM_SHARED((N, HALF), jnp.float32),
        ] + [pltpu.SemaphoreType.DMA] * 6,
    )(functools.partial(_agg_body, edge_split))


def _prep_body(x_ref, hist_ref, hs_ref, degv_ref):
    deg_out = hist_ref[0, 0, :] + hist_ref[1, 0, :]
    deg_in = hist_ref[0, 1, :] + hist_ref[1, 1, :]
    dinv_out = jax.lax.rsqrt(jnp.maximum(deg_out, 1.0))
    dinv_in = jax.lax.rsqrt(jnp.maximum(deg_in, 1.0))
    degv_ref[0, :] = dinv_out
    degv_ref[1, :] = dinv_in
    hs_ref[...] = x_ref[...] * dinv_out[:N, None]


def _tc_prep(features, hist):
    return pl.pallas_call(
        _prep_body,
        out_shape=(
            jax.ShapeDtypeStruct((N, DIN), jnp.float32),
            jax.ShapeDtypeStruct((2, NPAD), jnp.float32),
        ),
    )(features, hist)


BLK = 1024
NBLK = NPAD // BLK


def _mm_body(sum_mode, a0_ref, a1_ref, degv_ref, w_ref, b_ref, t_ref, stats_ref):
    i = pl.program_id(0)
    if sum_mode:
        a = a0_ref[...] + a1_ref[...]
    else:
        a = jnp.concatenate([a0_ref[...], a1_ref[...]], axis=1)
    din = degv_ref[1, pl.ds(i * BLK, BLK)]
    a = a * din[:, None]
    t = jnp.dot(a, w_ref[...], preferred_element_type=jnp.float32, precision=jax.lax.Precision.HIGHEST) + b_ref[...]
    t_ref[...] = t

    @pl.when(i == 0)
    def _():
        stats_ref[...] = jnp.zeros_like(stats_ref)
    rows = jax.lax.broadcasted_iota(jnp.int32, (BLK, 1), 0) + i * BLK
    tm = jnp.where(rows < N, t, 0.0)
    stats_ref[0, :] += jnp.sum(tm, axis=0)
    stats_ref[1, :] += jnp.sum(tm * tm, axis=0)


def _tc_matmul(agg0, agg1, degv, W, b, sum_mode):
    din, dout = W.shape
    return pl.pallas_call(
        functools.partial(_mm_body, sum_mode),
        grid=(NBLK,),
        in_specs=[
            pl.BlockSpec((BLK, HALF), lambda i: (i, 0)),
            pl.BlockSpec((BLK, HALF), lambda i: (i, 0)),
            pl.BlockSpec((2, NPAD), lambda i: (0, 0)),
            pl.BlockSpec((din, dout), lambda i: (0, 0)),
            pl.BlockSpec((dout,), lambda i: (0,)),
        ],
        out_specs=(
            pl.BlockSpec((BLK, dout), lambda i: (i, 0)),
            pl.BlockSpec((2, dout), lambda i: (0, 0)),
        ),
        out_shape=(
            jax.ShapeDtypeStruct((NPAD, dout), jnp.float32),
            jax.ShapeDtypeStruct((2, dout), jnp.float32),
        ),
    )(agg0, agg1, degv, W, b)


def _bn_body(t_ref, stats_ref, g_ref, be_ref, degv_ref, h0_ref, h1_ref):
    i = pl.program_id(0)
    mean = stats_ref[0, :] * (1.0 / N)
    var = stats_ref[1, :] * (1.0 / N) - mean * mean
    inv = jax.lax.rsqrt(var + EPS)
    y = (t_ref[...] - mean[None, :]) * (inv * g_ref[...])[None, :] + be_ref[...][None, :]
    y = jnp.maximum(y, 0.0)
    dout = degv_ref[0, pl.ds(i * BLK, BLK)]
    y = y * dout[:, None]
    h0_ref[...] = y[:, : DH // 2]
    h1_ref[...] = y[:, DH // 2:]


def _tc_bn_split(t, stats, g, be, degv):
    return pl.pallas_call(
        _bn_body,
        grid=(NBLK,),
        in_specs=[
            pl.BlockSpec((BLK, DH), lambda i: (i, 0)),
            pl.BlockSpec((2, DH), lambda i: (0, 0)),
            pl.BlockSpec((DH,), lambda i: (0,)),
            pl.BlockSpec((DH,), lambda i: (0,)),
            pl.BlockSpec((2, NPAD), lambda i: (0, 0)),
        ],
        out_specs=(
            pl.BlockSpec((BLK, DH // 2), lambda i: (i, 0)),
            pl.BlockSpec((BLK, DH // 2), lambda i: (i, 0)),
        ),
        out_shape=(
            jax.ShapeDtypeStruct((NPAD, DH // 2), jnp.float32),
            jax.ShapeDtypeStruct((NPAD, DH // 2), jnp.float32),
        ),
    )(t, stats, g, be, degv)


def _cls_body(t_ref, stats_ref, g_ref, be_ref, wc_ref, bc_ref, o_ref):
    mean = stats_ref[0, :] * (1.0 / N)
    var = stats_ref[1, :] * (1.0 / N) - mean * mean
    inv = jax.lax.rsqrt(var + EPS)
    y = (t_ref[...] - mean[None, :]) * (inv * g_ref[...])[None, :] + be_ref[...][None, :]
    y = jnp.maximum(y, 0.0)
    o_ref[...] = jnp.dot(y, wc_ref[...], preferred_element_type=jnp.float32, precision=jax.lax.Precision.HIGHEST) + bc_ref[...]


def _tc_classifier(t, stats, g, be, wc_pad, bc_pad):
    return pl.pallas_call(
        _cls_body,
        grid=(NBLK,),
        in_specs=[
            pl.BlockSpec((BLK, DH), lambda i: (i, 0)),
            pl.BlockSpec((2, DH), lambda i: (0, 0)),
            pl.BlockSpec((DH,), lambda i: (0,)),
            pl.BlockSpec((DH,), lambda i: (0,)),
            pl.BlockSpec((DH, 128), lambda i: (0, 0)),
            pl.BlockSpec((128,), lambda i: (0,)),
        ],
        out_specs=pl.BlockSpec((BLK, 128), lambda i: (i, 0)),
        out_shape=jax.ShapeDtypeStruct((NPAD, 128), jnp.float32),
    )(t, stats, g, be, wc_pad, bc_pad)


def kernel(features, edge_index, W1, b1, g1, be1, W2, b2, g2, be2,
           W3, b3, g3, be3, Wc, bc):
    ei = edge_index.reshape(2, ER, EC)

    hist = _make_deg_kernel()(ei)
    hs, degv = _tc_prep(features, hist)
    zer = jnp.zeros((632, HALF), jnp.float32)

    agg0, agg1 = _make_agg_kernel(True)(hs, hs, ei, zer)
    t1, st1 = _tc_matmul(agg0, agg1, degv, W1, b1, True)
    h0, h1 = _tc_bn_split(t1, st1, g1, be1, degv)

    agg0, agg1 = _make_agg_kernel(False)(h0, h1, ei, zer)
    t2, st2 = _tc_matmul(agg0, agg1, degv, W2, b2, False)
    h0, h1 = _tc_bn_split(t2, st2, g2, be2, degv)

    agg0, agg1 = _make_agg_kernel(False)(h0, h1, ei, zer)
    t3, st3 = _tc_matmul(agg0, agg1, degv, W3, b3, False)

    wc_pad = jnp.zeros((DH, 128), jnp.float32).at[:, :NCLS].set(Wc)
    bc_pad = jnp.zeros((128,), jnp.float32).at[:NCLS].set(bc)
    out = _tc_classifier(t3, st3, g3, be3, wc_pad, bc_pad)
    return out[:N, :NCLS]
